# baseline (speedup 1.0000x reference)
import jax
import jax.numpy as jnp
from jax import lax
from jax.experimental import pallas as pl
from jax.experimental.pallas import tpu as pltpu
from jax.experimental.pallas import tpu_sc as plsc

NUM_NODES = 10000
NUM_TRIPLES = 320000
NUM_RELATIONS = 1000
DIM = 128

L = 16
NW = 32
B_PER_W = NUM_TRIPLES // NW
CHUNK = 80
N_CHUNKS = B_PER_W // CHUNK
GROUPS = CHUNK // L
DIMW = DIM // 2
SLOTS = 2


def _distmult_body(z_hbm, g2l_hbm, heads_hbm, rels_hbm, tails_hbm, rel_hbm,
                   out_hbm, g2l_v, hds_v, rls_v, tls_v, rel_v,
                   idx_bufs, row_bufs, out_bufs, sems, osems):
    wid = lax.axis_index("s") * 2 + lax.axis_index("c")
    wbase = wid * B_PER_W

    pltpu.sync_copy(g2l_hbm, g2l_v)
    pltpu.sync_copy(heads_hbm.at[pl.ds(wbase, B_PER_W)], hds_v)
    pltpu.sync_copy(rels_hbm.at[pl.ds(wbase, B_PER_W)], rls_v)
    pltpu.sync_copy(tails_hbm.at[pl.ds(wbase, B_PER_W)], tls_v)
    pltpu.sync_copy(rel_hbm, rel_v)

    @pl.loop(0, B_PER_W // L, unroll=4)
    def _gmap(k):
        sl = pl.ds(k * L, L)
        hds_v[sl] = plsc.load_gather(g2l_v, [hds_v[sl]])
        tls_v[sl] = plsc.load_gather(g2l_v, [tls_v[sl]])

    def copies(s):
        hi, ti = idx_bufs[s]
        hrow, trow = row_bufs[s]
        return (pltpu.make_async_copy(z_hbm.at[hi], hrow, sems[s]),
                pltpu.make_async_copy(z_hbm.at[ti], trow, sems[s]))

    def start(ci, s):
        hi, ti = idx_bufs[s]
        for k in range(GROUPS):
            src = pl.ds(ci * CHUNK + k * L, L)
            dst = pl.ds(k * L, L)
            hi[dst] = hds_v[src]
            ti[dst] = tls_v[src]
        for cp in copies(s):
            cp.start()

    def drain(s):
        for cp in copies(s):
            cp.wait()

    def out_copy(ci, s):
        return pltpu.make_async_copy(
            out_bufs[s], out_hbm.at[pl.ds(wbase + ci * CHUNK, CHUNK)],
            osems[s])

    lane = lax.iota(jnp.int32, L)
    boff = (lane // 2) * 8
    himask = jnp.full((L,), -65536, dtype=jnp.int32)

    def compute(ci, s):
        hrow, trow = row_bufs[s]
        rows = [lane + (g * L) for g in range(GROUPS)]
        rids = [rls_v[pl.ds(ci * CHUNK + g * L, L)] for g in range(GROUPS)]
        zero = jnp.zeros((L,), jnp.float32)

        @pl.loop(0, DIMW,
                 init_carry=([(zero, zero) for _ in range(GROUPS)], boff),
                 unroll=8)
        def acc(kk, carry):
            cs, col = carry
            ncs = []
            for g in range(GROUPS):
                c1, c2 = cs[g]
                hb = plsc.bitcast(plsc.load_gather(hrow, [rows[g], col]),
                                  jnp.bfloat16)
                rb = plsc.bitcast(plsc.load_gather(rel_v, [rids[g], col]),
                                  jnp.bfloat16)
                tb = plsc.bitcast(plsc.load_gather(trow, [rows[g], col]),
                                  jnp.bfloat16)
                pw = plsc.bitcast(hb * rb * tb, jnp.int32)
                c1 = c1 + lax.bitcast_convert_type(pw << 16, jnp.float32)
                c2 = c2 + lax.bitcast_convert_type(pw & himask, jnp.float32)
                ncs.append((c1, c2))
            return ncs, (col + 1) & (DIMW - 1)

        for g in range(GROUPS):
            c1, c2 = acc[0][g]
            out_bufs[s][pl.ds(g * L, L)] = c1 + c2

    for c in range(SLOTS):
        start(c, c)

    @pl.loop(0, (N_CHUNKS - 1) // SLOTS)
    def ring(j):
        for m in range(SLOTS):
            c = j * SLOTS + m
            drain(m)

            @pl.when(c >= SLOTS)
            def _():
                out_copy(c - SLOTS, m).wait()

            compute(c, m)
            out_copy(c, m).start()

            @pl.when(c + SLOTS < N_CHUNKS)
            def _():
                start(c + SLOTS, m)

    last = N_CHUNKS - 1
    drain(last % SLOTS)
    out_copy(last - SLOTS, last % SLOTS).wait()
    compute(last, last % SLOTS)
    out_copy(last, last % SLOTS).start()
    out_copy(last - 1, (last - 1) % SLOTS).wait()
    out_copy(last, last % SLOTS).wait()


@jax.jit
def _distmult(z_local, g2l, heads, rels, tails, rel_emb):
    mesh = plsc.VectorSubcoreMesh(core_axis_name="c", subcore_axis_name="s")
    idx_t = pltpu.VMEM((CHUNK,), jnp.int32)
    row_t = pltpu.VMEM((CHUNK, DIMW), jnp.int32)
    kfn = pl.kernel(
        _distmult_body,
        mesh=mesh,
        compiler_params=pltpu.CompilerParams(needs_layout_passes=False,
                                             use_tc_tiling_on_sc=False),
        out_type=jax.ShapeDtypeStruct((NUM_TRIPLES,), jnp.float32),
        scratch_types=[
            pltpu.VMEM((NUM_NODES,), jnp.int32),
            pltpu.VMEM((B_PER_W,), jnp.int32),
            pltpu.VMEM((B_PER_W,), jnp.int32),
            pltpu.VMEM((B_PER_W,), jnp.int32),
            pltpu.VMEM((NUM_RELATIONS, DIMW), jnp.int32),
            [(idx_t, idx_t) for _ in range(SLOTS)],
            [(row_t, row_t) for _ in range(SLOTS)],
            [pltpu.VMEM((CHUNK,), jnp.float32) for _ in range(SLOTS)],
            [pltpu.SemaphoreType.DMA for _ in range(SLOTS)],
            [pltpu.SemaphoreType.DMA for _ in range(SLOTS)],
        ],
    )
    return kfn(z_local, g2l, heads, rels, tails, rel_emb)


def _pack_bf16(table):
    b = table.astype(jnp.bfloat16).reshape(table.shape[0], DIMW, 2)
    return lax.bitcast_convert_type(b, jnp.int32)


def kernel(z_local, global2local, heads, rels, tails, relation_emb):
    return _distmult(
        _pack_bf16(z_local),
        global2local.astype(jnp.int32),
        heads.astype(jnp.int32),
        rels.astype(jnp.int32),
        tails.astype(jnp.int32),
        _pack_bf16(relation_emb),
    )


# speedup vs baseline: 1.8503x; 1.8503x over previous
import jax
import jax.numpy as jnp
from jax import lax
from jax.experimental import pallas as pl
from jax.experimental.pallas import tpu as pltpu
from jax.experimental.pallas import tpu_sc as plsc

NUM_NODES = 10000
NUM_TRIPLES = 320000
NUM_RELATIONS = 1000
DIM = 128

L = 16
NW = 32
B_PER_W = NUM_TRIPLES // NW
CHUNK = 80
N_CHUNKS = B_PER_W // CHUNK
GROUPS = CHUNK // L
DIMW = DIM // 2
SLOTS = 2


def _distmult_body(z_hbm, g2l_hbm, heads_hbm, rels_hbm, tails_hbm, rel_hbm,
                   out_hbm, g2l_v, hds_v, rls_v, tls_v, rel_v,
                   idx_bufs, row_bufs, out_bufs, sems, osems):
    wid = lax.axis_index("s") * 2 + lax.axis_index("c")
    wbase = wid * B_PER_W

    pltpu.sync_copy(g2l_hbm, g2l_v)
    pltpu.sync_copy(heads_hbm.at[pl.ds(wbase, B_PER_W)], hds_v)
    pltpu.sync_copy(rels_hbm.at[pl.ds(wbase, B_PER_W)], rls_v)
    pltpu.sync_copy(tails_hbm.at[pl.ds(wbase, B_PER_W)], tls_v)
    pltpu.sync_copy(rel_hbm, rel_v)

    @pl.loop(0, B_PER_W // L, unroll=4)
    def _gmap(k):
        sl = pl.ds(k * L, L)
        hds_v[sl] = plsc.load_gather(g2l_v, [hds_v[sl]])
        tls_v[sl] = plsc.load_gather(g2l_v, [tls_v[sl]])

    def copies(s):
        hi, ti = idx_bufs[s]
        hrow, trow = row_bufs[s]
        return (pltpu.make_async_copy(z_hbm.at[hi], hrow, sems[s]),
                pltpu.make_async_copy(z_hbm.at[ti], trow, sems[s]))

    def start(ci, s):
        hi, ti = idx_bufs[s]
        for k in range(GROUPS):
            src = pl.ds(ci * CHUNK + k * L, L)
            dst = pl.ds(k * L, L)
            hi[dst] = hds_v[src]
            ti[dst] = tls_v[src]
        for cp in copies(s):
            cp.start()

    def drain(s):
        for cp in copies(s):
            cp.wait()

    def out_copy(ci, s):
        return pltpu.make_async_copy(
            out_bufs[s], out_hbm.at[pl.ds(wbase + ci * CHUNK, CHUNK)],
            osems[s])

    lane = lax.iota(jnp.int32, L)
    himask = jnp.full((L,), -65536, dtype=jnp.int32)

    def compute(ci, s):
        hrow, trow = row_bufs[s]
        rows = [lane + (g * L) for g in range(GROUPS)]
        rids = [rls_v[pl.ds(ci * CHUNK + g * L, L)] for g in range(GROUPS)]
        zero = jnp.zeros((L,), jnp.float32)

        @pl.loop(0, DIMW,
                 init_carry=([(zero, zero) for _ in range(GROUPS)], lane),
                 unroll=16)
        def acc(kk, carry):
            cs, col = carry
            ncs = []
            for g in range(GROUPS):
                c1, c2 = cs[g]
                hb = plsc.bitcast(plsc.load_gather(hrow, [rows[g], col]),
                                  jnp.bfloat16)
                rb = plsc.bitcast(plsc.load_gather(rel_v, [rids[g], col]),
                                  jnp.bfloat16)
                tb = plsc.bitcast(plsc.load_gather(trow, [rows[g], col]),
                                  jnp.bfloat16)
                pw = plsc.bitcast(hb * rb * tb, jnp.int32)
                c1 = c1 + lax.bitcast_convert_type(pw << 16, jnp.float32)
                c2 = c2 + lax.bitcast_convert_type(pw & himask, jnp.float32)
                ncs.append((c1, c2))
            return ncs, (col + 1) & (DIMW - 1)

        for g in range(GROUPS):
            c1, c2 = acc[0][g]
            out_bufs[s][pl.ds(g * L, L)] = c1 + c2

    for c in range(SLOTS):
        start(c, c)

    @pl.loop(0, (N_CHUNKS - 1) // SLOTS)
    def ring(j):
        for m in range(SLOTS):
            c = j * SLOTS + m
            drain(m)

            @pl.when(c >= SLOTS)
            def _():
                out_copy(c - SLOTS, m).wait()

            compute(c, m)
            out_copy(c, m).start()

            @pl.when(c + SLOTS < N_CHUNKS)
            def _():
                start(c + SLOTS, m)

    last = N_CHUNKS - 1
    drain(last % SLOTS)
    out_copy(last - SLOTS, last % SLOTS).wait()
    compute(last, last % SLOTS)
    out_copy(last, last % SLOTS).start()
    out_copy(last - 1, (last - 1) % SLOTS).wait()
    out_copy(last, last % SLOTS).wait()


@jax.jit
def _distmult(z_local, g2l, heads, rels, tails, rel_emb):
    mesh = plsc.VectorSubcoreMesh(core_axis_name="c", subcore_axis_name="s")
    idx_t = pltpu.VMEM((CHUNK,), jnp.int32)
    row_t = pltpu.VMEM((CHUNK, DIMW), jnp.int32)
    kfn = pl.kernel(
        _distmult_body,
        mesh=mesh,
        compiler_params=pltpu.CompilerParams(needs_layout_passes=False,
                                             use_tc_tiling_on_sc=False),
        out_type=jax.ShapeDtypeStruct((NUM_TRIPLES,), jnp.float32),
        scratch_types=[
            pltpu.VMEM((NUM_NODES,), jnp.int32),
            pltpu.VMEM((B_PER_W,), jnp.int32),
            pltpu.VMEM((B_PER_W,), jnp.int32),
            pltpu.VMEM((B_PER_W,), jnp.int32),
            pltpu.VMEM((NUM_RELATIONS, DIMW), jnp.int32),
            [(idx_t, idx_t) for _ in range(SLOTS)],
            [(row_t, row_t) for _ in range(SLOTS)],
            [pltpu.VMEM((CHUNK,), jnp.float32) for _ in range(SLOTS)],
            [pltpu.SemaphoreType.DMA for _ in range(SLOTS)],
            [pltpu.SemaphoreType.DMA for _ in range(SLOTS)],
        ],
    )
    return kfn(z_local, g2l, heads, rels, tails, rel_emb)


def _pack_bf16(table):
    b = table.astype(jnp.bfloat16).reshape(table.shape[0], DIMW, 2)
    return lax.bitcast_convert_type(b, jnp.int32)


def kernel(z_local, global2local, heads, rels, tails, relation_emb):
    return _distmult(
        _pack_bf16(z_local),
        global2local.astype(jnp.int32),
        heads.astype(jnp.int32),
        rels.astype(jnp.int32),
        tails.astype(jnp.int32),
        _pack_bf16(relation_emb),
    )
